# baseline (speedup 1.0000x reference)
import functools

import jax
import jax.numpy as jnp
from jax import lax
from jax.experimental import pallas as pl
from jax.experimental.pallas import tpu as pltpu
from jax.experimental.pallas import tpu_sc as plsc

_N = 10000
_E = 320000
_D = 128

_NC = 2
_NS = 16
_NW = _NC * _NS
_C = 128
_NCHUNK = 80
_EPW = _C * _NCHUNK
_EP = _EPW * _NW
_NP = 10240
_DUMMY = 10000
_RPT = _NP // _NS
_ZROWS = 128


def _sc_scatter(src3, dst3, x, zeros):
    mesh = plsc.VectorSubcoreMesh(core_axis_name="c", subcore_axis_name="s")

    @functools.partial(
        pl.kernel,
        mesh=mesh,
        out_type=jax.ShapeDtypeStruct((_NC, _NP, _D), jnp.float32),
        scratch_types=[
            pltpu.VMEM((_C,), jnp.int32),
            pltpu.VMEM((_C,), jnp.int32),
            pltpu.VMEM((_C,), jnp.int32),
            pltpu.VMEM((_C,), jnp.int32),
            pltpu.VMEM((_C, _D), jnp.float32),
            pltpu.VMEM((_C, _D), jnp.float32),
            pltpu.VMEM_SHARED((_NP, _D), jnp.float32),
            pltpu.SemaphoreType.DMA,
            pltpu.SemaphoreType.DMA,
            pltpu.SemaphoreType.DMA,
            pltpu.SemaphoreType.DMA,
            pltpu.SemaphoreType.DMA,
            pltpu.SemaphoreType.DMA,
            pltpu.SemaphoreType.DMA,
            pltpu.SemaphoreType.DMA,
            pltpu.SemaphoreType.DMA,
        ],
    )
    def k(src_hbm, dst_hbm, x_hbm, z_hbm, out_hbm,
          sidx0, sidx1, didx0, didx1, rows0, rows1, acc,
          gs0, gs1, ss0, ss1, fs0, fs1, fd0, fd1, zs):
        cid = lax.axis_index("c")
        sid = lax.axis_index("s")
        wid = sid * _NC + cid
        base_row = sid * _RPT

        def sfetch(j, buf, sem):
            return pltpu.async_copy(src_hbm.at[wid, j], buf, sem)

        def dfetch(j, buf, sem):
            return pltpu.async_copy(dst_hbm.at[wid, j], buf, sem)

        def gather(buf, sbuf, sem):
            return pltpu.async_copy(x_hbm.at[sbuf], buf, sem)

        def scat(buf, dbuf, sem):
            return pltpu.async_copy(buf, acc.at[dbuf], sem, add=True)

        zcps = [pltpu.async_copy(
            z_hbm, acc.at[pl.ds(base_row + j * _ZROWS, _ZROWS)], zs)
            for j in range(_RPT // _ZROWS)]
        sf = sfetch(0, sidx0, fs0)
        df = dfetch(0, didx0, fd0)
        sf.wait()
        df.wait()
        g0 = gather(rows0, sidx0, gs0)
        sfetch(1, sidx1, fs1)
        dfetch(1, didx1, fd1)
        g0.wait()
        sfetch(2, sidx0, fs0)
        for z in zcps:
            z.wait()
        plsc.subcore_barrier()
        scat(rows0, didx0, ss0)

        sc0_w = pltpu.make_async_copy(rows0, acc.at[didx0], ss0)
        fs0_w = pltpu.make_async_copy(src_hbm.at[wid, 0], sidx0, fs0)
        fs1_w = pltpu.make_async_copy(src_hbm.at[wid, 0], sidx1, fs1)
        fd1_w = pltpu.make_async_copy(dst_hbm.at[wid, 0], didx1, fd1)

        def pair(i, carry):
            b = 2 * i + 2
            na = jnp.minimum(2 * i + 3, _NCHUNK - 1)
            nb = jnp.minimum(2 * i + 4, _NCHUNK - 1)
            fs1_w.wait()
            ga = gather(rows1, sidx1, gs1)
            sc0_w.wait()
            dfb = dfetch(b, didx0, fd0)
            fs0_w.wait()
            ga.wait()
            sfetch(na, sidx1, fs1)
            fd1_w.wait()
            sa = scat(rows1, didx1, ss1)
            gb = gather(rows0, sidx0, gs0)
            dfb.wait()
            sa.wait()
            dfetch(na, didx1, fd1)
            gb.wait()
            sfetch(nb, sidx0, fs0)
            scat(rows0, didx0, ss0)
            return carry
        lax.fori_loop(0, (_NCHUNK - 2) // 2, pair, 0)

        fs1_w.wait()
        ga = gather(rows1, sidx1, gs1)
        sc0_w.wait()
        fs0_w.wait()
        ga.wait()
        fd1_w.wait()
        scat(rows1, didx1, ss1).wait()
        plsc.subcore_barrier()

        pltpu.sync_copy(acc.at[pl.ds(base_row, _RPT)],
                        out_hbm.at[cid, pl.ds(base_row, _RPT)])

    return k(src3, dst3, x, zeros)


def _tc_add(p0, p1):
    blk = 1000

    def body(a_ref, b_ref, o_ref):
        o_ref[...] = a_ref[...] + b_ref[...]

    return pl.pallas_call(
        body,
        grid=(_N // blk,),
        in_specs=[pl.BlockSpec((blk, _D), lambda i: (i, 0)),
                  pl.BlockSpec((blk, _D), lambda i: (i, 0))],
        out_specs=pl.BlockSpec((blk, _D), lambda i: (i, 0)),
        out_shape=jax.ShapeDtypeStruct((_N, _D), jnp.float32),
    )(p0, p1)


def kernel(edge_index, x):
    ei = edge_index.astype(jnp.int32)
    npad = _EP - _E
    src = jnp.concatenate([ei[0], jnp.zeros((npad,), jnp.int32)])
    dst = jnp.concatenate([ei[1], jnp.full((npad,), _DUMMY, jnp.int32)])
    src3 = src.reshape(_NW, _NCHUNK, _C)
    dst3 = dst.reshape(_NW, _NCHUNK, _C)
    zeros = jnp.zeros((_ZROWS, _D), jnp.float32)
    partials = _sc_scatter(src3, dst3, x, zeros)
    return _tc_add(partials[0], partials[1])


# speedup vs baseline: 3.1861x; 3.1861x over previous
import functools

import jax
import jax.numpy as jnp
from jax import lax
from jax.experimental import pallas as pl
from jax.experimental.pallas import tpu as pltpu
from jax.experimental.pallas import tpu_sc as plsc

_N = 10000
_E = 320000
_D = 128

_NC = 2
_NS = 16
_NW = _NC * _NS
_EPW = _E // _NW
_C = 80
_NCHUNK = _EPW // _C
_NP = 10240
_RPT = _NP // _NS
_ZROWS = 128


def _sc_scatter(src3, dst3, x, zeros):
    mesh = plsc.VectorSubcoreMesh(core_axis_name="c", subcore_axis_name="s")

    @functools.partial(
        pl.kernel,
        mesh=mesh,
        out_type=jax.ShapeDtypeStruct((_NC, _NP, _D), jnp.float32),
        scratch_types=[
            pltpu.VMEM((_NCHUNK, _C), jnp.int32),
            pltpu.VMEM((_C,), jnp.int32),
            pltpu.VMEM((_C,), jnp.int32),
            pltpu.VMEM((_C, _D), jnp.float32),
            pltpu.VMEM((_C, _D), jnp.float32),
            pltpu.VMEM_SHARED((_NP, _D), jnp.float32),
            pltpu.SemaphoreType.DMA,
            pltpu.SemaphoreType.DMA,
            pltpu.SemaphoreType.DMA,
            pltpu.SemaphoreType.DMA,
            pltpu.SemaphoreType.DMA,
            pltpu.SemaphoreType.DMA,
            pltpu.SemaphoreType.DMA,
        ],
    )
    def k(src_hbm, dst_hbm, x_hbm, z_hbm, out_hbm,
          sidx, didx0, didx1, rows0, rows1, acc, g0, g1, s0, s1, i0, i1, zs):
        cid = lax.axis_index("c")
        sid = lax.axis_index("s")
        wid = sid * _NC + cid
        base_row = sid * _RPT

        def gather(j, buf, sem):
            return pltpu.async_copy(x_hbm.at[sidx.at[j]], buf, sem)

        def scat(dbuf, buf, sem):
            return pltpu.async_copy(buf, acc.at[dbuf], sem, add=True)

        def dfetch(j, dbuf, sem):
            return pltpu.async_copy(dst_hbm.at[wid, j], dbuf, sem)

        zcps = [pltpu.async_copy(
            z_hbm, acc.at[pl.ds(base_row + j * _ZROWS, _ZROWS)], zs)
            for j in range(_RPT // _ZROWS)]
        df0 = dfetch(0, didx0, i0)
        pltpu.sync_copy(src_hbm.at[wid], sidx)
        g0p = gather(0, rows0, g0)
        df1 = dfetch(1, didx1, i1)
        df0.wait()
        g0p.wait()
        for z in zcps:
            z.wait()
        plsc.subcore_barrier()
        sc0 = scat(didx0, rows0, s0)

        def pair(i, carry):
            a = 2 * i + 1
            b = 2 * i + 2
            ga = gather(a, rows1, g1)
            sc0.wait()
            dfb = dfetch(b, didx0, i0)
            gb = gather(b, rows0, g0)
            ga.wait()
            df1.wait()
            sa = scat(didx1, rows1, s1)
            gb.wait()
            dfb.wait()
            sa.wait()
            nxt = jnp.minimum(a + 2, _NCHUNK - 1)
            dfetch(nxt, didx1, i1)
            scat(didx0, rows0, s0)
            return carry
        lax.fori_loop(0, (_NCHUNK - 1) // 2, pair, 0)
        df1.wait()
        sc0.wait()
        plsc.subcore_barrier()

        pltpu.sync_copy(acc.at[pl.ds(base_row, _RPT)],
                        out_hbm.at[cid, pl.ds(base_row, _RPT)])

    return k(src3, dst3, x, zeros)


def _tc_add(p0, p1):
    blk = 1000

    def body(a_ref, b_ref, o_ref):
        o_ref[...] = a_ref[...] + b_ref[...]

    return pl.pallas_call(
        body,
        grid=(_N // blk,),
        in_specs=[pl.BlockSpec((blk, _D), lambda i: (i, 0)),
                  pl.BlockSpec((blk, _D), lambda i: (i, 0))],
        out_specs=pl.BlockSpec((blk, _D), lambda i: (i, 0)),
        out_shape=jax.ShapeDtypeStruct((_N, _D), jnp.float32),
    )(p0, p1)


def kernel(edge_index, x):
    ei = edge_index.astype(jnp.int32)
    src3 = ei[0].reshape(_NW, _NCHUNK, _C)
    dst3 = ei[1].reshape(_NW, _NCHUNK, _C)
    zeros = jnp.zeros((_ZROWS, _D), jnp.float32)
    partials = _sc_scatter(src3, dst3, x, zeros)
    return _tc_add(partials[0], partials[1])
